# baseline (speedup 1.0000x reference)
import functools

import jax
import jax.numpy as jnp
import numpy as np
from jax.experimental import pallas as pl
from jax.experimental.pallas import tpu as pltpu

_TIME_STEPS = 1000
_BETA_START = 0.0001
_BETA_END = 0.02

_betas = np.linspace(_BETA_START, _BETA_END, _TIME_STEPS, dtype=np.float64)
_alphas_cumprod = np.cumprod(1.0 - _betas)
_TABLE = np.concatenate(
    [np.sqrt(_alphas_cumprod), np.sqrt(1.0 - _alphas_cumprod)]
).astype(np.float32)

_BB = 32


def _fma_body(time_ref, tab_ref, x_ref, n_ref, o_ref):
    g = pl.program_id(0)
    for i in range(_BB):
        t = time_ref[g * _BB + i]
        a = tab_ref[t]
        c = tab_ref[_TIME_STEPS + t]
        o_ref[i] = a * x_ref[i] + c * n_ref[i]


@jax.jit
def kernel(x, time, noise):
    b, ch, h, w = x.shape
    rows = ch * h * w // 128
    x3 = x.reshape(b, rows, 128)
    n3 = noise.reshape(b, rows, 128)
    tab = jnp.asarray(_TABLE)

    grid = b // _BB
    spec = pl.BlockSpec((_BB, rows, 128), lambda g, *_: (g, 0, 0))
    out = pl.pallas_call(
        _fma_body,
        grid_spec=pltpu.PrefetchScalarGridSpec(
            num_scalar_prefetch=2,
            grid=(grid,),
            in_specs=[spec, spec],
            out_specs=spec,
        ),
        out_shape=jax.ShapeDtypeStruct((b, rows, 128), jnp.float32),
    )(time, tab, x3, n3)
    return out.reshape(x.shape)


# speedup vs baseline: 1.0080x; 1.0080x over previous
import functools

import jax
import jax.numpy as jnp
import numpy as np
from jax.experimental import pallas as pl
from jax.experimental.pallas import tpu as pltpu

_TIME_STEPS = 1000
_BETA_START = 0.0001
_BETA_END = 0.02

_betas = np.linspace(_BETA_START, _BETA_END, _TIME_STEPS, dtype=np.float64)
_alphas_cumprod = np.cumprod(1.0 - _betas)
_TABLE = np.concatenate(
    [np.sqrt(_alphas_cumprod), np.sqrt(1.0 - _alphas_cumprod)]
).astype(np.float32)

_BB = 16


def _fma_body(time_ref, tab_ref, x_ref, n_ref, o_ref):
    g = pl.program_id(0)
    for i in range(_BB):
        t = time_ref[g * _BB + i]
        a = tab_ref[t]
        c = tab_ref[_TIME_STEPS + t]
        o_ref[i] = a * x_ref[i] + c * n_ref[i]


@jax.jit
def kernel(x, time, noise):
    b, ch, h, w = x.shape
    rows = ch * h * w // 128
    x3 = x.reshape(b, rows, 128)
    n3 = noise.reshape(b, rows, 128)
    tab = jnp.asarray(_TABLE)

    grid = b // _BB
    spec = pl.BlockSpec((_BB, rows, 128), lambda g, *_: (g, 0, 0))
    out = pl.pallas_call(
        _fma_body,
        grid_spec=pltpu.PrefetchScalarGridSpec(
            num_scalar_prefetch=2,
            grid=(grid,),
            in_specs=[spec, spec],
            out_specs=spec,
        ),
        out_shape=jax.ShapeDtypeStruct((b, rows, 128), jnp.float32),
    )(time, tab, x3, n3)
    return out.reshape(x.shape)
